# baseline (speedup 1.0000x reference)
import jax
import jax.numpy as jnp
from jax import lax
from jax.experimental import pallas as pl
from jax.experimental.pallas import tpu as pltpu
from jax.experimental.pallas import tpu_sc as plsc

_K = 8192
_D = 256
_TM = 128


def _prep_body(cb_ref, cn_ref, cbbf_ref, iota_ref):
    cb = cb_ref[...]
    cn_col = jnp.sum(cb * cb, axis=1, keepdims=True)
    cn_ref[...] = cn_col.T
    cbbf_ref[...] = cb.astype(jnp.bfloat16)
    iota_ref[...] = lax.broadcasted_iota(jnp.int32, (1, _K), 1).astype(
        jnp.float32)


def _prep(codebook):
    return pl.pallas_call(
        _prep_body,
        out_shape=[
            jax.ShapeDtypeStruct((1, _K), jnp.float32),
            jax.ShapeDtypeStruct((_K, _D), jnp.bfloat16),
            jax.ShapeDtypeStruct((1, _K), jnp.float32),
        ],
    )(codebook)


def _vq_body(z_ref, cn_ref, cbbf_ref, iota_ref,
             idx_ref, oh_ref, ppl_ref, acc_ref):
    step = pl.program_id(0)
    nsteps = pl.num_programs(0)

    @pl.when(step == 0)
    def _():
        acc_ref[...] = jnp.zeros_like(acc_ref)

    z = z_ref[...]
    zn = jnp.sum(z * z, axis=1, keepdims=True)
    zbf = (z * jnp.float32(-2.0)).astype(jnp.bfloat16)
    dot2 = lax.dot_general(
        zbf, cbbf_ref[...], (((1,), (1,)), ((), ())),
        preferred_element_type=jnp.float32,
    )
    d = (zn + dot2) + cn_ref[...]
    idx_i = jnp.argmin(d, axis=1)[:, None]
    idx_ref[...] = idx_i
    iota_f = jnp.broadcast_to(iota_ref[...], (_TM, _K))
    idx_f = idx_i.astype(jnp.float32)
    oh = jnp.where(iota_f == idx_f, jnp.float32(1), jnp.float32(0))
    oh_ref[...] = oh
    ohbf = oh.astype(jnp.bfloat16)
    ones_row = jnp.ones((8, _TM), jnp.bfloat16)
    acc_ref[...] = acc_ref[...] + lax.dot_general(
        ones_row, ohbf, (((1,), (0,)), ((), ())),
        preferred_element_type=jnp.float32,
    )

    @pl.when(step == nsteps - 1)
    def _():
        total = jnp.float32(nsteps * _TM)
        p = acc_ref[0:1] / total
        s = jnp.sum(p * jnp.log(p + 1e-10), axis=1, keepdims=True)
        ppl_ref[...] = jnp.exp(-s)


def _tc_quantize(z_flat, codebook):
    m = z_flat.shape[0]
    grid = (m // _TM,)
    cn, cbbf, iota = _prep(codebook)
    return pl.pallas_call(
        _vq_body,
        grid=grid,
        in_specs=[
            pl.BlockSpec((_TM, _D), lambda i: (i, 0)),
            pl.BlockSpec((1, _K), lambda i: (0, 0)),
            pl.BlockSpec((_K, _D), lambda i: (0, 0)),
            pl.BlockSpec((1, _K), lambda i: (0, 0)),
        ],
        out_specs=[
            pl.BlockSpec((_TM, 1), lambda i: (i, 0)),
            pl.BlockSpec((_TM, _K), lambda i: (i, 0)),
            pl.BlockSpec((1, 1), lambda i: (0, 0)),
        ],
        out_shape=[
            jax.ShapeDtypeStruct((m, 1), jnp.int32),
            jax.ShapeDtypeStruct((m, _K), jnp.float32),
            jax.ShapeDtypeStruct((1, 1), jnp.float32),
        ],
        scratch_shapes=[
            pltpu.VMEM((8, _K), jnp.float32),
        ],
    )(z_flat, cn, cbbf, iota)


_GATHER_W = 128


def _sc_gather(codebook, indices):
    n = indices.shape[0]
    idx2 = indices.reshape(1, n)
    mesh = plsc.VectorSubcoreMesh(core_axis_name="core", subcore_axis_name="subcore")

    @pl.kernel(
        out_type=jax.ShapeDtypeStruct((n, _D), codebook.dtype),
        mesh=mesh,
    )
    def gather_kernel(cb_hbm, i_hbm, o_hbm):
        def body(i_vmem, o_vmem):
            pltpu.sync_copy(cb_hbm.at[i_vmem.at[0]], o_vmem)

        pltpu.emit_pipeline(
            body,
            grid=(n // _GATHER_W,),
            in_specs=[pl.BlockSpec((1, _GATHER_W), index_map=lambda i: (0, i))],
            out_specs=[pl.BlockSpec((_GATHER_W, _D), index_map=lambda i: (i, 0))],
            core_axis_name=("core", "subcore"),
            dimension_semantics=(pltpu.PARALLEL,),
        )(i_hbm, o_hbm)

    return gather_kernel(codebook, idx2)


def kernel(z, codebook):
    b, n, d = z.shape
    z_flat = z.reshape(b * n, d)
    idx, min_encodings, ppl = _tc_quantize(z_flat, codebook)
    e_indices = idx.reshape(-1)
    z_quantized = _sc_gather(codebook, e_indices).reshape(b, n, d)
    return z_quantized, min_encodings, e_indices, ppl[0, 0]


# speedup vs baseline: 1.3389x; 1.3389x over previous
import jax
import jax.numpy as jnp
from jax import lax
from jax.experimental import pallas as pl
from jax.experimental.pallas import tpu as pltpu
from jax.experimental.pallas import tpu_sc as plsc

_K = 8192
_D = 256
_TM = 512


def _prep_body(cb_ref, cn_ref, cbbf_ref, iota_ref):
    cb = cb_ref[...]
    cn_col = jnp.sum(cb * cb, axis=1, keepdims=True)
    cn_ref[...] = cn_col.T
    cbbf_ref[...] = cb.astype(jnp.bfloat16)
    iota_ref[...] = lax.broadcasted_iota(jnp.int32, (1, _K), 1).astype(
        jnp.float32)


def _prep(codebook):
    return pl.pallas_call(
        _prep_body,
        out_shape=[
            jax.ShapeDtypeStruct((1, _K), jnp.float32),
            jax.ShapeDtypeStruct((_K, _D), jnp.bfloat16),
            jax.ShapeDtypeStruct((1, _K), jnp.float32),
        ],
    )(codebook)


def _vq_body(z_ref, cn_ref, cbbf_ref, iota_ref,
             idx_ref, oh_ref, ppl_ref, acc_ref):
    step = pl.program_id(0)
    nsteps = pl.num_programs(0)

    @pl.when(step == 0)
    def _():
        acc_ref[...] = jnp.zeros_like(acc_ref)

    z = z_ref[...]
    zn = jnp.sum(z * z, axis=1, keepdims=True)
    zbf = (z * jnp.float32(-2.0)).astype(jnp.bfloat16)
    dot2 = lax.dot_general(
        zbf, cbbf_ref[...], (((1,), (1,)), ((), ())),
        preferred_element_type=jnp.float32,
    )
    d = (zn + dot2) + cn_ref[...]
    idx_i = jnp.argmin(d, axis=1)[:, None]
    idx_ref[...] = idx_i
    iota_f = jnp.broadcast_to(iota_ref[...], (_TM, _K))
    idx_f = idx_i.astype(jnp.float32)
    oh = jnp.where(iota_f == idx_f, jnp.float32(1), jnp.float32(0))
    oh_ref[...] = oh
    ohbf = oh.astype(jnp.bfloat16)
    ones_row = jnp.ones((8, _TM), jnp.bfloat16)
    acc_ref[...] = acc_ref[...] + lax.dot_general(
        ones_row, ohbf, (((1,), (0,)), ((), ())),
        preferred_element_type=jnp.float32,
    )

    @pl.when(step == nsteps - 1)
    def _():
        total = jnp.float32(nsteps * _TM)
        p = acc_ref[0:1] / total
        s = jnp.sum(p * jnp.log(p + 1e-10), axis=1, keepdims=True)
        ppl_ref[...] = jnp.exp(-s)


def _tc_quantize(z_flat, codebook):
    m = z_flat.shape[0]
    grid = (m // _TM,)
    cn, cbbf, iota = _prep(codebook)
    return pl.pallas_call(
        _vq_body,
        grid=grid,
        in_specs=[
            pl.BlockSpec((_TM, _D), lambda i: (i, 0)),
            pl.BlockSpec((1, _K), lambda i: (0, 0)),
            pl.BlockSpec((_K, _D), lambda i: (0, 0)),
            pl.BlockSpec((1, _K), lambda i: (0, 0)),
        ],
        out_specs=[
            pl.BlockSpec((_TM, 1), lambda i: (i, 0)),
            pl.BlockSpec((_TM, _K), lambda i: (i, 0)),
            pl.BlockSpec((1, 1), lambda i: (0, 0)),
        ],
        out_shape=[
            jax.ShapeDtypeStruct((m, 1), jnp.int32),
            jax.ShapeDtypeStruct((m, _K), jnp.float32),
            jax.ShapeDtypeStruct((1, 1), jnp.float32),
        ],
        scratch_shapes=[
            pltpu.VMEM((8, _K), jnp.float32),
        ],
    )(z_flat, cn, cbbf, iota)


_GATHER_W = 128


def _sc_gather(codebook, indices):
    n = indices.shape[0]
    idx2 = indices.reshape(1, n)
    mesh = plsc.VectorSubcoreMesh(core_axis_name="core", subcore_axis_name="subcore")

    @pl.kernel(
        out_type=jax.ShapeDtypeStruct((n, _D), codebook.dtype),
        mesh=mesh,
    )
    def gather_kernel(cb_hbm, i_hbm, o_hbm):
        def body(i_vmem, o_vmem):
            pltpu.sync_copy(cb_hbm.at[i_vmem.at[0]], o_vmem)

        pltpu.emit_pipeline(
            body,
            grid=(n // _GATHER_W,),
            in_specs=[pl.BlockSpec((1, _GATHER_W), index_map=lambda i: (0, i))],
            out_specs=[pl.BlockSpec((_GATHER_W, _D), index_map=lambda i: (i, 0))],
            core_axis_name=("core", "subcore"),
            dimension_semantics=(pltpu.PARALLEL,),
        )(i_hbm, o_hbm)

    return gather_kernel(codebook, idx2)


def kernel(z, codebook):
    b, n, d = z.shape
    z_flat = z.reshape(b * n, d)
    idx, min_encodings, ppl = _tc_quantize(z_flat, codebook)
    e_indices = idx.reshape(-1)
    z_quantized = _sc_gather(codebook, e_indices).reshape(b, n, d)
    return z_quantized, min_encodings, e_indices, ppl[0, 0]
